# baseline (speedup 1.0000x reference)
import functools

import jax
import jax.numpy as jnp
from jax import lax
from jax.experimental import pallas as pl
from jax.experimental.pallas import tpu as pltpu
from jax.experimental.pallas import tpu_sc as plsc

NC = 2
NS = 16
NW = NC * NS
L = 16

P = 2000


def _sqrt16(x):
    i = lax.bitcast_convert_type(x, jnp.int32)
    i = 0x5F3759DF - lax.shift_right_logical(i, 1)
    y = lax.bitcast_convert_type(i, jnp.float32)
    for _ in range(2):
        y = y * (1.5 - 0.5 * x * y * y)
    return x * y


def _make_body(n_pairs):
    pairs_per_w = n_pairs // NW
    n_chunks = pairs_per_w // P

    assert n_chunks % 2 == 0

    def body(tbl_hbm, pairs_hbm, r_hbm, d_hbm,
             a_i, a_j, idx_i0, idx_j0, idx_i1, idx_j1,
             rows_i0, rows_j0, rows_i1, rows_j1,
             out_r0, out_d0, out_r1, out_d1,
             sem_i0, sem_j0, sem_i1, sem_j1, sem_out):
        wid = lax.axis_index("s") * NC + lax.axis_index("c")
        w_base = wid * pairs_per_w

        def stage(k, idx_i, idx_j, rows_i, rows_j, sem_i, sem_j):
            base = w_base + k * P
            pltpu.sync_copy(pairs_hbm.at[pl.ds(base, P)], a_i)
            pltpu.sync_copy(pairs_hbm.at[pl.ds(n_pairs + base, P)], a_j)

            def build_idx(v, carry2):
                ai3 = a_i[pl.ds(v * L, L)] * 3
                aj3 = a_j[pl.ds(v * L, L)] * 3
                for c in range(3):
                    idx_i[pl.ds(c * P + v * L, L)] = ai3 + c
                    idx_j[pl.ds(c * P + v * L, L)] = aj3 + c
                return carry2

            lax.fori_loop(0, P // L, build_idx, 0, unroll=4)
            pltpu.async_copy(tbl_hbm.at[idx_i], rows_i, sem_i)
            pltpu.async_copy(tbl_hbm.at[idx_j], rows_j, sem_j)

        def wait_gathers(idx_i, idx_j, rows_i, rows_j, sem_i, sem_j):
            pltpu.make_async_copy(tbl_hbm.at[idx_i], rows_i, sem_i).wait()
            pltpu.make_async_copy(tbl_hbm.at[idx_j], rows_j, sem_j).wait()

        def compute_chunk(k, rows_i, rows_j, out_r, out_d):
            base = w_base + k * P

            def compute(g, carry2):
                o = g * L
                dx = rows_j[pl.ds(o, L)] - rows_i[pl.ds(o, L)]
                dy = rows_j[pl.ds(P + o, L)] - rows_i[pl.ds(P + o, L)]
                dz = rows_j[pl.ds(2 * P + o, L)] - rows_i[pl.ds(2 * P + o, L)]
                out_r[pl.ds(o, L)] = dx
                out_r[pl.ds(P + o, L)] = dy
                out_r[pl.ds(2 * P + o, L)] = dz
                out_d[pl.ds(o, L)] = _sqrt16(dx * dx + dy * dy + dz * dz)
                return carry2

            lax.fori_loop(0, P // L, compute, 0, unroll=4)
            for c in range(3):
                pltpu.sync_copy(out_r.at[pl.ds(c * P, P)],
                                r_hbm.at[pl.ds(c * n_pairs + base, P)])
            pltpu.sync_copy(out_d, d_hbm.at[pl.ds(base, P)])

        ii0, ij0, ri0, rj0, or0, od0, si0, sj0 = (
            idx_i0, idx_j0, rows_i0, rows_j0, out_r0, out_d0, sem_i0, sem_j0)
        ii1, ij1, ri1, rj1, or1, od1, si1, sj1 = (
            idx_i1, idx_j1, rows_i1, rows_j1, out_r1, out_d1, sem_i1, sem_j1)

        stage(0, ii0, ij0, ri0, rj0, si0, sj0)

        def pair_body(m, carry):
            k0 = 2 * m
            stage(k0 + 1, ii1, ij1, ri1, rj1, si1, sj1)
            wait_gathers(ii0, ij0, ri0, rj0, si0, sj0)
            compute_chunk(k0, ri0, rj0, or0, od0)
            nxt = jnp.where(k0 + 2 < n_chunks, k0 + 2, k0)
            stage(nxt, ii0, ij0, ri0, rj0, si0, sj0)
            wait_gathers(ii1, ij1, ri1, rj1, si1, sj1)
            compute_chunk(k0 + 1, ri1, rj1, or1, od1)
            return carry

        lax.fori_loop(0, n_chunks // 2, pair_body, 0, unroll=False)
        wait_gathers(ii0, ij0, ri0, rj0, si0, sj0)

    return body


@jax.jit
def kernel(positions, pair_indices):
    n_pairs = pair_indices.shape[1]
    tbl = positions.reshape(-1)
    pairs_flat = pair_indices.reshape(-1)

    body = _make_body(n_pairs)
    run = pl.kernel(
        body,
        out_type=(
            jax.ShapeDtypeStruct((3 * n_pairs,), jnp.float32),
            jax.ShapeDtypeStruct((n_pairs,), jnp.float32),
        ),
        mesh=plsc.VectorSubcoreMesh(core_axis_name="c", subcore_axis_name="s",
                                    num_cores=NC, num_subcores=NS),
        scratch_types=(
            pltpu.VMEM((P,), jnp.int32),
            pltpu.VMEM((P,), jnp.int32),
            pltpu.VMEM((P * 3,), jnp.int32),
            pltpu.VMEM((P * 3,), jnp.int32),
            pltpu.VMEM((P * 3,), jnp.int32),
            pltpu.VMEM((P * 3,), jnp.int32),
            pltpu.VMEM((P * 3,), jnp.float32),
            pltpu.VMEM((P * 3,), jnp.float32),
            pltpu.VMEM((P * 3,), jnp.float32),
            pltpu.VMEM((P * 3,), jnp.float32),
            pltpu.VMEM((P * 3,), jnp.float32),
            pltpu.VMEM((P,), jnp.float32),
            pltpu.VMEM((P * 3,), jnp.float32),
            pltpu.VMEM((P,), jnp.float32),
            pltpu.SemaphoreType.DMA,
            pltpu.SemaphoreType.DMA,
            pltpu.SemaphoreType.DMA,
            pltpu.SemaphoreType.DMA,
            pltpu.SemaphoreType.DMA,
        ),
        compiler_params=pltpu.CompilerParams(use_tc_tiling_on_sc=False),
    )
    r_planes, d_flat = run(tbl, pairs_flat)
    r_ij = r_planes.reshape(3, n_pairs).T
    return (d_flat.reshape(n_pairs, 1), r_ij)


# speedup vs baseline: 1.7708x; 1.7708x over previous
import functools

import jax
import jax.numpy as jnp
from jax import lax
from jax.experimental import pallas as pl
from jax.experimental.pallas import tpu as pltpu
from jax.experimental.pallas import tpu_sc as plsc

NC = 2
NS = 16
NW = NC * NS
L = 16

P = 2000


def _sqrt16(x):
    i = lax.bitcast_convert_type(x, jnp.int32)
    i = 0x5F3759DF - lax.shift_right_logical(i, 1)
    y = lax.bitcast_convert_type(i, jnp.float32)
    for _ in range(2):
        y = y * (1.5 - 0.5 * x * y * y)
    return x * y


def _make_body(n_pairs):
    pairs_per_w = n_pairs // NW
    n_chunks = pairs_per_w // P

    assert n_chunks % 2 == 0

    def body(tbl_hbm, pairs_hbm, r_hbm, d_hbm,
             a_i, a_j, idx_i0, idx_j0, idx_i1, idx_j1,
             rows_i0, rows_j0, rows_i1, rows_j1,
             out_r0, out_d0, out_r1, out_d1,
             sem_i0, sem_j0, sem_i1, sem_j1, sem_out):
        wid = lax.axis_index("s") * NC + lax.axis_index("c")
        w_base = wid * pairs_per_w

        def stage(k, idx_i, idx_j, rows_i, rows_j, sem_i, sem_j):
            base = w_base + k * P
            pltpu.sync_copy(pairs_hbm.at[pl.ds(base, P)], a_i)
            pltpu.sync_copy(pairs_hbm.at[pl.ds(n_pairs + base, P)], a_j)

            def build_idx(v, carry2):
                ai3 = a_i[pl.ds(v * L, L)] * 3
                aj3 = a_j[pl.ds(v * L, L)] * 3
                for c in range(3):
                    idx_i[pl.ds(c * P + v * L, L)] = ai3 + c
                    idx_j[pl.ds(c * P + v * L, L)] = aj3 + c
                return carry2

            lax.fori_loop(0, P // L, build_idx, 0, unroll=4)
            pltpu.async_copy(tbl_hbm.at[idx_i], rows_i, sem_i)
            pltpu.async_copy(tbl_hbm.at[idx_j], rows_j, sem_j)

        def wait_gathers(idx_i, idx_j, rows_i, rows_j, sem_i, sem_j):
            pltpu.make_async_copy(tbl_hbm.at[idx_i], rows_i, sem_i).wait()
            pltpu.make_async_copy(tbl_hbm.at[idx_j], rows_j, sem_j).wait()

        def compute_chunk(k, rows_i, rows_j, out_r, out_d):
            base = w_base + k * P

            def compute(g, carry2):
                o = g * L
                dx = rows_j[pl.ds(o, L)] - rows_i[pl.ds(o, L)]
                dy = rows_j[pl.ds(P + o, L)] - rows_i[pl.ds(P + o, L)]
                dz = rows_j[pl.ds(2 * P + o, L)] - rows_i[pl.ds(2 * P + o, L)]
                out_r[pl.ds(o, L)] = dx
                out_r[pl.ds(P + o, L)] = dy
                out_r[pl.ds(2 * P + o, L)] = dz
                out_d[pl.ds(o, L)] = _sqrt16(dx * dx + dy * dy + dz * dz)
                return carry2

            lax.fori_loop(0, P // L, compute, 0, unroll=4)
            for c in range(3):
                pltpu.sync_copy(out_r.at[pl.ds(c * P, P)],
                                r_hbm.at[pl.ds(c * n_pairs + base, P)])
            pltpu.sync_copy(out_d, d_hbm.at[pl.ds(base, P)])

        ii0, ij0, ri0, rj0, or0, od0, si0, sj0 = (
            idx_i0, idx_j0, rows_i0, rows_j0, out_r0, out_d0, sem_i0, sem_j0)
        ii1, ij1, ri1, rj1, or1, od1, si1, sj1 = (
            idx_i1, idx_j1, rows_i1, rows_j1, out_r1, out_d1, sem_i1, sem_j1)

        stage(0, ii0, ij0, ri0, rj0, si0, sj0)

        def pair_body(m, carry):
            k0 = 2 * m
            stage(k0 + 1, ii1, ij1, ri1, rj1, si1, sj1)
            wait_gathers(ii0, ij0, ri0, rj0, si0, sj0)
            compute_chunk(k0, ri0, rj0, or0, od0)
            nxt = jnp.where(k0 + 2 < n_chunks, k0 + 2, k0)
            stage(nxt, ii0, ij0, ri0, rj0, si0, sj0)
            wait_gathers(ii1, ij1, ri1, rj1, si1, sj1)
            compute_chunk(k0 + 1, ri1, rj1, or1, od1)
            return carry

        lax.fori_loop(0, n_chunks // 2, pair_body, 0, unroll=False)
        wait_gathers(ii0, ij0, ri0, rj0, si0, sj0)

    return body


@jax.jit
def kernel(positions, pair_indices):
    n_pairs = pair_indices.shape[1]
    tbl = positions.reshape(-1)
    pairs_flat = pair_indices.reshape(-1)

    body = _make_body(n_pairs)
    run = pl.kernel(
        body,
        out_type=(
            jax.ShapeDtypeStruct((3 * n_pairs,), jnp.float32),
            jax.ShapeDtypeStruct((n_pairs,), jnp.float32),
        ),
        mesh=plsc.VectorSubcoreMesh(core_axis_name="c", subcore_axis_name="s",
                                    num_cores=NC, num_subcores=NS),
        scratch_types=(
            pltpu.VMEM((P,), jnp.int32),
            pltpu.VMEM((P,), jnp.int32),
            pltpu.VMEM((P * 3,), jnp.int32),
            pltpu.VMEM((P * 3,), jnp.int32),
            pltpu.VMEM((P * 3,), jnp.int32),
            pltpu.VMEM((P * 3,), jnp.int32),
            pltpu.VMEM((P * 3,), jnp.float32),
            pltpu.VMEM((P * 3,), jnp.float32),
            pltpu.VMEM((P * 3,), jnp.float32),
            pltpu.VMEM((P * 3,), jnp.float32),
            pltpu.VMEM((P * 3,), jnp.float32),
            pltpu.VMEM((P,), jnp.float32),
            pltpu.VMEM((P * 3,), jnp.float32),
            pltpu.VMEM((P,), jnp.float32),
            pltpu.SemaphoreType.DMA,
            pltpu.SemaphoreType.DMA,
            pltpu.SemaphoreType.DMA,
            pltpu.SemaphoreType.DMA,
            pltpu.SemaphoreType.DMA,
        ),
        compiler_params=pltpu.CompilerParams(use_tc_tiling_on_sc=False),
    )
    r_planes, d_flat = run(tbl, pairs_flat)
    r_ij = jnp.stack([r_planes[0:n_pairs],
                      r_planes[n_pairs:2 * n_pairs],
                      r_planes[2 * n_pairs:3 * n_pairs]], axis=1)
    return (d_flat.reshape(n_pairs, 1), r_ij)


# speedup vs baseline: 1.7830x; 1.0069x over previous
import functools

import jax
import jax.numpy as jnp
from jax import lax
from jax.experimental import pallas as pl
from jax.experimental.pallas import tpu as pltpu
from jax.experimental.pallas import tpu_sc as plsc

NC = 2
NS = 16
NW = NC * NS
L = 16

P = 4000


def _sqrt16(x):
    i = lax.bitcast_convert_type(x, jnp.int32)
    i = 0x5F3759DF - lax.shift_right_logical(i, 1)
    y = lax.bitcast_convert_type(i, jnp.float32)
    for _ in range(2):
        y = y * (1.5 - 0.5 * x * y * y)
    return x * y


def _make_body(n_pairs):
    pairs_per_w = n_pairs // NW
    n_chunks = pairs_per_w // P


    def body(tbl_hbm, pairs_hbm, r_hbm, d_hbm,
             a_i, a_j, idx_i0, idx_j0, idx_i1, idx_j1,
             rows_i0, rows_j0, rows_i1, rows_j1,
             out_r0, out_d0,
             sem_i0, sem_j0, sem_i1, sem_j1, sem_out):
        wid = lax.axis_index("s") * NC + lax.axis_index("c")
        w_base = wid * pairs_per_w

        def stage(k, idx_i, idx_j, rows_i, rows_j, sem_i, sem_j):
            base = w_base + k * P
            pltpu.sync_copy(pairs_hbm.at[pl.ds(base, P)], a_i)
            pltpu.sync_copy(pairs_hbm.at[pl.ds(n_pairs + base, P)], a_j)

            def build_idx(v, carry2):
                ai3 = a_i[pl.ds(v * L, L)] * 3
                aj3 = a_j[pl.ds(v * L, L)] * 3
                for c in range(3):
                    idx_i[pl.ds(c * P + v * L, L)] = ai3 + c
                    idx_j[pl.ds(c * P + v * L, L)] = aj3 + c
                return carry2

            lax.fori_loop(0, P // L, build_idx, 0, unroll=4)
            pltpu.async_copy(tbl_hbm.at[idx_i], rows_i, sem_i)
            pltpu.async_copy(tbl_hbm.at[idx_j], rows_j, sem_j)

        def wait_gathers(idx_i, idx_j, rows_i, rows_j, sem_i, sem_j):
            pltpu.make_async_copy(tbl_hbm.at[idx_i], rows_i, sem_i).wait()
            pltpu.make_async_copy(tbl_hbm.at[idx_j], rows_j, sem_j).wait()

        def compute_chunk(k, rows_i, rows_j, out_r, out_d):
            base = w_base + k * P

            def compute(g, carry2):
                o = g * L
                dx = rows_j[pl.ds(o, L)] - rows_i[pl.ds(o, L)]
                dy = rows_j[pl.ds(P + o, L)] - rows_i[pl.ds(P + o, L)]
                dz = rows_j[pl.ds(2 * P + o, L)] - rows_i[pl.ds(2 * P + o, L)]
                out_r[pl.ds(o, L)] = dx
                out_r[pl.ds(P + o, L)] = dy
                out_r[pl.ds(2 * P + o, L)] = dz
                out_d[pl.ds(o, L)] = _sqrt16(dx * dx + dy * dy + dz * dz)
                return carry2

            lax.fori_loop(0, P // L, compute, 0, unroll=4)
            for c in range(3):
                pltpu.sync_copy(out_r.at[pl.ds(c * P, P)],
                                r_hbm.at[pl.ds(c * n_pairs + base, P)])
            pltpu.sync_copy(out_d, d_hbm.at[pl.ds(base, P)])

        ii0, ij0, ri0, rj0, or0, od0, si0, sj0 = (
            idx_i0, idx_j0, rows_i0, rows_j0, out_r0, out_d0, sem_i0, sem_j0)
        ii1, ij1, ri1, rj1, or1, od1, si1, sj1 = (
            idx_i1, idx_j1, rows_i1, rows_j1, out_r0, out_d0, sem_i1, sem_j1)

        stage(0, ii0, ij0, ri0, rj0, si0, sj0)

        def pair_body(m, carry):
            k0 = 2 * m
            stage(k0 + 1, ii1, ij1, ri1, rj1, si1, sj1)
            wait_gathers(ii0, ij0, ri0, rj0, si0, sj0)
            compute_chunk(k0, ri0, rj0, or0, od0)
            nxt = jnp.where(k0 + 2 < n_chunks, k0 + 2, k0)
            stage(nxt, ii0, ij0, ri0, rj0, si0, sj0)
            wait_gathers(ii1, ij1, ri1, rj1, si1, sj1)
            compute_chunk(k0 + 1, ri1, rj1, or1, od1)
            return carry

        lax.fori_loop(0, n_chunks // 2, pair_body, 0, unroll=False)
        wait_gathers(ii0, ij0, ri0, rj0, si0, sj0)
        if n_chunks % 2 == 1:
            compute_chunk(n_chunks - 1, ri0, rj0, or0, od0)

    return body


@jax.jit
def kernel(positions, pair_indices):
    n_pairs = pair_indices.shape[1]
    tbl = positions.reshape(-1)
    pairs_flat = pair_indices.reshape(-1)

    body = _make_body(n_pairs)
    run = pl.kernel(
        body,
        out_type=(
            jax.ShapeDtypeStruct((3 * n_pairs,), jnp.float32),
            jax.ShapeDtypeStruct((n_pairs,), jnp.float32),
        ),
        mesh=plsc.VectorSubcoreMesh(core_axis_name="c", subcore_axis_name="s",
                                    num_cores=NC, num_subcores=NS),
        scratch_types=(
            pltpu.VMEM((P,), jnp.int32),
            pltpu.VMEM((P,), jnp.int32),
            pltpu.VMEM((P * 3,), jnp.int32),
            pltpu.VMEM((P * 3,), jnp.int32),
            pltpu.VMEM((P * 3,), jnp.int32),
            pltpu.VMEM((P * 3,), jnp.int32),
            pltpu.VMEM((P * 3,), jnp.float32),
            pltpu.VMEM((P * 3,), jnp.float32),
            pltpu.VMEM((P * 3,), jnp.float32),
            pltpu.VMEM((P * 3,), jnp.float32),
            pltpu.VMEM((P * 3,), jnp.float32),
            pltpu.VMEM((P,), jnp.float32),
            pltpu.SemaphoreType.DMA,
            pltpu.SemaphoreType.DMA,
            pltpu.SemaphoreType.DMA,
            pltpu.SemaphoreType.DMA,
            pltpu.SemaphoreType.DMA,
        ),
        compiler_params=pltpu.CompilerParams(use_tc_tiling_on_sc=False),
    )
    r_planes, d_flat = run(tbl, pairs_flat)
    r_ij = jnp.stack([r_planes[0:n_pairs],
                      r_planes[n_pairs:2 * n_pairs],
                      r_planes[2 * n_pairs:3 * n_pairs]], axis=1)
    return (d_flat.reshape(n_pairs, 1), r_ij)


# speedup vs baseline: 1.7842x; 1.0007x over previous
import jax
import jax.numpy as jnp
from jax import lax
from jax.experimental import pallas as pl
from jax.experimental.pallas import tpu as pltpu
from jax.experimental.pallas import tpu_sc as plsc

NC = 2
NS = 16
NW = NC * NS
L = 16

P = 4000


def _sqrt16(x):
    i = lax.bitcast_convert_type(x, jnp.int32)
    i = 0x5F3759DF - lax.shift_right_logical(i, 1)
    y = lax.bitcast_convert_type(i, jnp.float32)
    for _ in range(2):
        y = y * (1.5 - 0.5 * x * y * y)
    return x * y


def _make_body(n_pairs):
    pairs_per_w = n_pairs // NW
    n_chunks = pairs_per_w // P


    def body(tbl_hbm, pairs_hbm, r_hbm, d_hbm,
             a_i, a_j, idx_i0, idx_j0, idx_i1, idx_j1,
             rows_i0, rows_j0, rows_i1, rows_j1,
             out_r0, out_d0,
             sem_i0, sem_j0, sem_i1, sem_j1, sem_out):
        wid = lax.axis_index("s") * NC + lax.axis_index("c")
        w_base = wid * pairs_per_w

        def stage(k, idx_i, idx_j, rows_i, rows_j, sem_i, sem_j):
            base = w_base + k * P
            pltpu.sync_copy(pairs_hbm.at[pl.ds(base, P)], a_i)
            pltpu.sync_copy(pairs_hbm.at[pl.ds(n_pairs + base, P)], a_j)

            def build_idx(v, carry2):
                ai3 = a_i[pl.ds(v * L, L)] * 3
                aj3 = a_j[pl.ds(v * L, L)] * 3
                for c in range(3):
                    idx_i[pl.ds(c * P + v * L, L)] = ai3 + c
                    idx_j[pl.ds(c * P + v * L, L)] = aj3 + c
                return carry2

            lax.fori_loop(0, P // L, build_idx, 0, unroll=4)
            pltpu.async_copy(tbl_hbm.at[idx_i], rows_i, sem_i)
            pltpu.async_copy(tbl_hbm.at[idx_j], rows_j, sem_j)

        def wait_gathers(idx_i, idx_j, rows_i, rows_j, sem_i, sem_j):
            pltpu.make_async_copy(tbl_hbm.at[idx_i], rows_i, sem_i).wait()
            pltpu.make_async_copy(tbl_hbm.at[idx_j], rows_j, sem_j).wait()

        def compute_chunk(k, rows_i, rows_j, out_r, out_d):
            base = w_base + k * P

            def compute(g, carry2):
                o = g * L
                dx = rows_j[pl.ds(o, L)] - rows_i[pl.ds(o, L)]
                dy = rows_j[pl.ds(P + o, L)] - rows_i[pl.ds(P + o, L)]
                dz = rows_j[pl.ds(2 * P + o, L)] - rows_i[pl.ds(2 * P + o, L)]
                out_r[pl.ds(o, L)] = dx
                out_r[pl.ds(P + o, L)] = dy
                out_r[pl.ds(2 * P + o, L)] = dz
                out_d[pl.ds(o, L)] = _sqrt16(dx * dx + dy * dy + dz * dz)
                return carry2

            lax.fori_loop(0, P // L, compute, 0, unroll=4)
            for c in range(3):
                pltpu.sync_copy(out_r.at[pl.ds(c * P, P)],
                                r_hbm.at[pl.ds(c * n_pairs + base, P)])
            pltpu.sync_copy(out_d, d_hbm.at[pl.ds(base, P)])

        ii0, ij0, ri0, rj0, or0, od0, si0, sj0 = (
            idx_i0, idx_j0, rows_i0, rows_j0, out_r0, out_d0, sem_i0, sem_j0)
        ii1, ij1, ri1, rj1, or1, od1, si1, sj1 = (
            idx_i1, idx_j1, rows_i1, rows_j1, out_r0, out_d0, sem_i1, sem_j1)

        stage(0, ii0, ij0, ri0, rj0, si0, sj0)

        def pair_body(m, carry):
            k0 = 2 * m
            stage(k0 + 1, ii1, ij1, ri1, rj1, si1, sj1)
            wait_gathers(ii0, ij0, ri0, rj0, si0, sj0)
            compute_chunk(k0, ri0, rj0, or0, od0)
            nxt = jnp.where(k0 + 2 < n_chunks, k0 + 2, k0)
            stage(nxt, ii0, ij0, ri0, rj0, si0, sj0)
            wait_gathers(ii1, ij1, ri1, rj1, si1, sj1)
            compute_chunk(k0 + 1, ri1, rj1, or1, od1)
            return carry

        lax.fori_loop(0, n_chunks // 2, pair_body, 0, unroll=False)
        wait_gathers(ii0, ij0, ri0, rj0, si0, sj0)
        if n_chunks % 2 == 1:
            compute_chunk(n_chunks - 1, ri0, rj0, or0, od0)

    return body


@jax.jit
def kernel(positions, pair_indices):
    n_pairs = pair_indices.shape[1]
    tbl = positions.reshape(-1)
    pairs_flat = pair_indices.reshape(-1)

    body = _make_body(n_pairs)
    run = pl.kernel(
        body,
        out_type=(
            jax.ShapeDtypeStruct((3 * n_pairs,), jnp.float32),
            jax.ShapeDtypeStruct((n_pairs,), jnp.float32),
        ),
        mesh=plsc.VectorSubcoreMesh(core_axis_name="c", subcore_axis_name="s",
                                    num_cores=NC, num_subcores=NS),
        scratch_types=(
            pltpu.VMEM((P,), jnp.int32),
            pltpu.VMEM((P,), jnp.int32),
            pltpu.VMEM((P * 3,), jnp.int32),
            pltpu.VMEM((P * 3,), jnp.int32),
            pltpu.VMEM((P * 3,), jnp.int32),
            pltpu.VMEM((P * 3,), jnp.int32),
            pltpu.VMEM((P * 3,), jnp.float32),
            pltpu.VMEM((P * 3,), jnp.float32),
            pltpu.VMEM((P * 3,), jnp.float32),
            pltpu.VMEM((P * 3,), jnp.float32),
            pltpu.VMEM((P * 3,), jnp.float32),
            pltpu.VMEM((P,), jnp.float32),
            pltpu.SemaphoreType.DMA,
            pltpu.SemaphoreType.DMA,
            pltpu.SemaphoreType.DMA,
            pltpu.SemaphoreType.DMA,
            pltpu.SemaphoreType.DMA,
        ),
        compiler_params=pltpu.CompilerParams(use_tc_tiling_on_sc=False),
    )
    r_planes, d_flat = run(tbl, pairs_flat)
    r_ij = jnp.stack([r_planes[0:n_pairs],
                      r_planes[n_pairs:2 * n_pairs],
                      r_planes[2 * n_pairs:3 * n_pairs]], axis=1)
    return (d_flat.reshape(n_pairs, 1), r_ij)
